# baseline (device time: 78891 ns/iter reference)
import jax
import jax.numpy as jnp
from jax import lax
from jax.experimental import pallas as pl
from jax.experimental.pallas import tpu as pltpu

N_DEV = 4


def _ring_allreduce_sum(p):
    M, N = p.shape

    def body(p_ref, out_ref, comm_ref, send_sems, recv_sems):
        my = lax.axis_index("i")
        left = lax.rem(my + (N_DEV - 1), N_DEV)
        right = lax.rem(my + 1, N_DEV)

        barrier_sem = pltpu.get_barrier_semaphore()
        for nbr in (left, right):
            pl.semaphore_signal(
                barrier_sem, inc=1,
                device_id=(nbr,), device_id_type=pl.DeviceIdType.MESH,
            )
        pl.semaphore_wait(barrier_sem, 2)

        out_ref[...] = p_ref[...].astype(jnp.float32)
        comm_ref[0, :, :] = p_ref[...]

        for h in range(N_DEV - 1):
            s = h % 2
            r = (h + 1) % 2
            rdma = pltpu.make_async_remote_copy(
                src_ref=comm_ref.at[s],
                dst_ref=comm_ref.at[r],
                send_sem=send_sems.at[s],
                recv_sem=recv_sems.at[r],
                device_id=(right,),
                device_id_type=pl.DeviceIdType.MESH,
            )
            rdma.start()
            rdma.wait()
            out_ref[...] += comm_ref[r, :, :].astype(jnp.float32)

    return pl.pallas_call(
        body,
        out_shape=jax.ShapeDtypeStruct((M, N), jnp.float32),
        in_specs=[pl.BlockSpec(memory_space=pltpu.VMEM)],
        out_specs=pl.BlockSpec(memory_space=pltpu.VMEM),
        scratch_shapes=[
            pltpu.VMEM((2, M, N), p.dtype),
            pltpu.SemaphoreType.DMA((2,)),
            pltpu.SemaphoreType.DMA((2,)),
        ],
        compiler_params=pltpu.CompilerParams(collective_id=0),
    )(p)


def kernel(x, Wq, K_ext, V_ext, Wo):
    B, Sq, d_model = x.shape
    _, Skv, H_loc, Dh = K_ext.shape
    d_loc = H_loc * Dh

    my = lax.axis_index("i")

    Wq_loc = lax.dynamic_slice_in_dim(Wq, my * d_loc, d_loc, axis=1)
    Wo_loc = lax.dynamic_slice_in_dim(Wo, my * d_loc, d_loc, axis=0)

    bf16 = jnp.bfloat16
    Q = (x.astype(bf16) @ Wq_loc.astype(bf16)).reshape(B, Sq, H_loc, Dh)

    qb = (jnp.arange(Sq) // 64)[:, None]
    kb = (jnp.arange(Skv) // 64)[None, :]
    mask = (qb == kb) | (kb == 0) | ((qb + kb) % 3 == 0)

    scores = jnp.einsum(
        "bihd,bjhd->bhij", Q, K_ext.astype(bf16),
        preferred_element_type=jnp.float32,
    ) * 0.125
    scores = jnp.where(mask[None, None], scores, -1e9)
    w = jax.nn.softmax(scores, axis=-1)

    ctx = jnp.einsum(
        "bhij,bjhd->bihd", w.astype(bf16), V_ext.astype(bf16),
        preferred_element_type=jnp.float32,
    ).reshape(B, Sq, d_loc)

    partial = (ctx.astype(bf16) @ Wo_loc.astype(bf16)).astype(bf16)

    out = _ring_allreduce_sum(partial.reshape(B * Sq, d_model))
    return out.reshape(B, Sq, d_model)


# device time: 44949 ns/iter; 1.7551x vs baseline; 1.7551x over previous
import jax
import jax.numpy as jnp
from jax import lax
from jax.experimental import pallas as pl
from jax.experimental.pallas import tpu as pltpu

N_DEV = 4


def _direct_allreduce_sum(p4):
    _, M, N = p4.shape

    def body(p_ref, out_ref, red_ref, rs_buf, ag_buf,
             rs_send, rs_recv, ag_send, ag_recv):
        my = lax.axis_index("i")

        barrier_sem = pltpu.get_barrier_semaphore()
        for r in (1, 2, 3):
            pl.semaphore_signal(
                barrier_sem, inc=1,
                device_id=((my + r) % N_DEV,),
                device_id_type=pl.DeviceIdType.MESH,
            )
        pl.semaphore_wait(barrier_sem, 3)

        rs_desc = []
        for r in (1, 2, 3):
            tgt = (my + r) % N_DEV
            d = pltpu.make_async_remote_copy(
                src_ref=p_ref.at[tgt],
                dst_ref=rs_buf.at[3 - r],
                send_sem=rs_send.at[r - 1],
                recv_sem=rs_recv.at[3 - r],
                device_id=(tgt,),
                device_id_type=pl.DeviceIdType.MESH,
            )
            d.start()
            rs_desc.append(d)

        acc = p_ref[my].astype(jnp.float32)
        for s in range(3):
            recv = pltpu.make_async_remote_copy(
                src_ref=rs_buf.at[s], dst_ref=rs_buf.at[s],
                send_sem=rs_send.at[0], recv_sem=rs_recv.at[s],
                device_id=(my,), device_id_type=pl.DeviceIdType.MESH,
            )
            recv.wait_recv()
            acc = acc + rs_buf[s].astype(jnp.float32)

        red_ref[...] = acc.astype(out_ref.dtype)
        out_ref[pl.ds(my * M, M), :] = red_ref[...]

        ag_desc = []
        for r in (1, 2, 3):
            tgt = (my + r) % N_DEV
            d = pltpu.make_async_remote_copy(
                src_ref=red_ref,
                dst_ref=ag_buf.at[3 - r],
                send_sem=ag_send.at[r - 1],
                recv_sem=ag_recv.at[3 - r],
                device_id=(tgt,),
                device_id_type=pl.DeviceIdType.MESH,
            )
            d.start()
            ag_desc.append(d)

        for d in rs_desc:
            d.wait_send()

        for q in (1, 2, 3):
            sender = (my + q) % N_DEV
            recv = pltpu.make_async_remote_copy(
                src_ref=ag_buf.at[q - 1], dst_ref=ag_buf.at[q - 1],
                send_sem=ag_send.at[0], recv_sem=ag_recv.at[q - 1],
                device_id=(my,), device_id_type=pl.DeviceIdType.MESH,
            )
            recv.wait_recv()
            out_ref[pl.ds(sender * M, M), :] = ag_buf[q - 1]

        for d in ag_desc:
            d.wait_send()

    return pl.pallas_call(
        body,
        out_shape=jax.ShapeDtypeStruct((N_DEV * M, N), p4.dtype),
        in_specs=[pl.BlockSpec(memory_space=pltpu.VMEM)],
        out_specs=pl.BlockSpec(memory_space=pltpu.VMEM),
        scratch_shapes=[
            pltpu.VMEM((M, N), p4.dtype),
            pltpu.VMEM((3, M, N), p4.dtype),
            pltpu.VMEM((3, M, N), p4.dtype),
            pltpu.SemaphoreType.DMA((3,)),
            pltpu.SemaphoreType.DMA((3,)),
            pltpu.SemaphoreType.DMA((3,)),
            pltpu.SemaphoreType.DMA((3,)),
        ],
        compiler_params=pltpu.CompilerParams(collective_id=0),
    )(p4)


def kernel(x, Wq, K_ext, V_ext, Wo):
    B, Sq, d_model = x.shape
    _, Skv, H_loc, Dh = K_ext.shape
    d_loc = H_loc * Dh

    my = lax.axis_index("i")

    Wq_loc = lax.dynamic_slice_in_dim(Wq, my * d_loc, d_loc, axis=1)
    Wo_loc = lax.dynamic_slice_in_dim(Wo, my * d_loc, d_loc, axis=0)

    bf16 = jnp.bfloat16
    Q = (x.astype(bf16) @ Wq_loc.astype(bf16)).reshape(B, Sq, H_loc, Dh)

    qb = (jnp.arange(Sq) // 64)[:, None]
    kb = (jnp.arange(Skv) // 64)[None, :]
    mask = (qb == kb) | (kb == 0) | ((qb + kb) % 3 == 0)

    scores = jnp.einsum(
        "bihd,bjhd->bhij", Q, K_ext.astype(bf16),
        preferred_element_type=jnp.float32,
    ) * 0.125
    scores = jnp.where(mask[None, None], scores, -1e9)
    w = jax.nn.softmax(scores, axis=-1)

    ctx = jnp.einsum(
        "bhij,bjhd->bihd", w.astype(bf16), V_ext.astype(bf16),
        preferred_element_type=jnp.float32,
    ).reshape(B, Sq, d_loc)

    partial = (ctx.astype(bf16) @ Wo_loc.astype(bf16)).astype(bf16)

    M = (B * Sq) // N_DEV
    out = _direct_allreduce_sum(partial.reshape(N_DEV, M, d_model))
    return out.reshape(B, Sq, d_model).astype(jnp.float32)


# device time: 40502 ns/iter; 1.9478x vs baseline; 1.1098x over previous
import jax
import jax.numpy as jnp
from jax import lax
from jax.experimental import pallas as pl
from jax.experimental.pallas import tpu as pltpu

N_DEV = 4


def kernel(x, Wq, K_ext, V_ext, Wo):
    B, Sq, d_model = x.shape
    _, Skv, H_loc, Dh = K_ext.shape
    d_loc = H_loc * Dh
    M = (B * Sq) // N_DEV
    Sq_half = Sq // 2

    bf16 = jnp.bfloat16
    my_out = lax.axis_index("i")

    Wq_loc = lax.dynamic_slice_in_dim(Wq, my_out * d_loc, d_loc, axis=1)
    Wo_loc = lax.dynamic_slice_in_dim(Wo, my_out * d_loc, d_loc, axis=0)

    xb = x.reshape(B * Sq, d_model).astype(bf16)
    K2 = K_ext.reshape(B, Skv, d_loc).astype(bf16)
    V2 = V_ext.reshape(B, Skv, d_loc).astype(bf16)

    def body(x_ref, wq_ref, k_ref, v_ref, wo_ref, out_ref,
             send_buf, red_ref, rs_buf, ag_buf,
             rs_send, rs_recv, ag_send, ag_recv):
        my = lax.axis_index("i")

        barrier_sem = pltpu.get_barrier_semaphore()
        for r in (1, 2, 3):
            pl.semaphore_signal(
                barrier_sem, inc=1,
                device_id=((my + r) % N_DEV,),
                device_id_type=pl.DeviceIdType.MESH,
            )
        pl.semaphore_wait(barrier_sem, 3)

        def compute_quarter(t):
            x_q = x_ref[pl.ds(t * M, M), :]
            q_q = jnp.dot(
                x_q, wq_ref[...], preferred_element_type=jnp.float32
            ).astype(bf16)
            b = t // 2
            kb_full = k_ref[b]
            vb_full = v_ref[b]
            off = (t % 2) * Sq_half
            qi = off + lax.broadcasted_iota(jnp.int32, (M, Skv), 0)
            kj = lax.broadcasted_iota(jnp.int32, (M, Skv), 1)
            qblk = qi // 64
            kblk = kj // 64
            mask = (qblk == kblk) | (kblk == 0) | ((qblk + kblk) % 3 == 0)
            ctx_parts = []
            for h in range(H_loc):
                cols = slice(h * Dh, (h + 1) * Dh)
                s = lax.dot_general(
                    q_q[:, cols], kb_full[:, cols],
                    (((1,), (1,)), ((), ())),
                    preferred_element_type=jnp.float32,
                ) * 0.125
                s = jnp.where(mask, s, -1e9)
                m_ = jnp.max(s, axis=1, keepdims=True)
                e = jnp.exp(s - m_)
                w = (e / jnp.sum(e, axis=1, keepdims=True)).astype(bf16)
                ctx_parts.append(jnp.dot(
                    w, vb_full[:, cols], preferred_element_type=jnp.float32
                ))
            ctx = jnp.concatenate(ctx_parts, axis=1).astype(bf16)
            return jnp.dot(
                ctx, wo_ref[...], preferred_element_type=jnp.float32
            )

        rs_desc = []
        for r in (1, 2, 3):
            tgt = (my + r) % N_DEV
            send_buf[r - 1, :, :] = compute_quarter(tgt).astype(bf16)
            d = pltpu.make_async_remote_copy(
                src_ref=send_buf.at[r - 1],
                dst_ref=rs_buf.at[3 - r],
                send_sem=rs_send.at[r - 1],
                recv_sem=rs_recv.at[3 - r],
                device_id=(tgt,),
                device_id_type=pl.DeviceIdType.MESH,
            )
            d.start()
            rs_desc.append(d)

        acc = compute_quarter(my)
        for s_ in range(3):
            recv = pltpu.make_async_remote_copy(
                src_ref=rs_buf.at[s_], dst_ref=rs_buf.at[s_],
                send_sem=rs_send.at[0], recv_sem=rs_recv.at[s_],
                device_id=(my,), device_id_type=pl.DeviceIdType.MESH,
            )
            recv.wait_recv()
            acc = acc + rs_buf[s_].astype(jnp.float32)

        red_ref[...] = acc.astype(bf16)
        out_ref[pl.ds(my * M, M), :] = red_ref[...]

        ag_desc = []
        for r in (1, 2, 3):
            tgt = (my + r) % N_DEV
            d = pltpu.make_async_remote_copy(
                src_ref=red_ref,
                dst_ref=ag_buf.at[3 - r],
                send_sem=ag_send.at[r - 1],
                recv_sem=ag_recv.at[3 - r],
                device_id=(tgt,),
                device_id_type=pl.DeviceIdType.MESH,
            )
            d.start()
            ag_desc.append(d)

        for d in rs_desc:
            d.wait_send()

        for q in (1, 2, 3):
            sender = (my + q) % N_DEV
            recv = pltpu.make_async_remote_copy(
                src_ref=ag_buf.at[q - 1], dst_ref=ag_buf.at[q - 1],
                send_sem=ag_send.at[0], recv_sem=ag_recv.at[q - 1],
                device_id=(my,), device_id_type=pl.DeviceIdType.MESH,
            )
            recv.wait_recv()
            out_ref[pl.ds(sender * M, M), :] = ag_buf[q - 1]

        for d in ag_desc:
            d.wait_send()

    out = pl.pallas_call(
        body,
        out_shape=jax.ShapeDtypeStruct((B * Sq, d_model), bf16),
        in_specs=[pl.BlockSpec(memory_space=pltpu.VMEM)] * 5,
        out_specs=pl.BlockSpec(memory_space=pltpu.VMEM),
        scratch_shapes=[
            pltpu.VMEM((3, M, d_model), bf16),
            pltpu.VMEM((M, d_model), bf16),
            pltpu.VMEM((3, M, d_model), bf16),
            pltpu.VMEM((3, M, d_model), bf16),
            pltpu.SemaphoreType.DMA((3,)),
            pltpu.SemaphoreType.DMA((3,)),
            pltpu.SemaphoreType.DMA((3,)),
            pltpu.SemaphoreType.DMA((3,)),
        ],
        compiler_params=pltpu.CompilerParams(collective_id=0),
    )(xb, Wq_loc.astype(bf16), K2, V2, Wo_loc.astype(bf16))

    return out.reshape(B, Sq, d_model)


# device time: 37221 ns/iter; 2.1195x vs baseline; 1.0881x over previous
import jax
import jax.numpy as jnp
import numpy as np
from jax import lax
from jax.experimental import pallas as pl
from jax.experimental.pallas import tpu as pltpu

N_DEV = 4


def _mask_bias(Sq, Skv):
    qb = (np.arange(Sq) // 64)[:, None]
    kb = (np.arange(Skv) // 64)[None, :]
    mask = (qb == kb) | (kb == 0) | ((qb + kb) % 3 == 0)
    bias = np.where(mask, 0.0, -30000.0).astype(np.float32)
    return bias.reshape(2, Sq // 2, Skv)


def kernel(x, Wq, K_ext, V_ext, Wo):
    B, Sq, d_model = x.shape
    _, Skv, H_loc, Dh = K_ext.shape
    d_loc = H_loc * Dh
    M = (B * Sq) // N_DEV

    bf16 = jnp.bfloat16
    my_out = lax.axis_index("i")

    Wq_loc = lax.dynamic_slice_in_dim(Wq, my_out * d_loc, d_loc, axis=1)
    Wo_loc = lax.dynamic_slice_in_dim(Wo, my_out * d_loc, d_loc, axis=0)

    xb = x.reshape(B * Sq, d_model).astype(bf16)
    K2 = K_ext.reshape(B, Skv, d_loc).astype(bf16)
    V2 = V_ext.reshape(B, Skv, d_loc).astype(bf16)
    bias = jnp.asarray(_mask_bias(Sq, Skv))

    def body(x_ref, wq_ref, k_ref, v_ref, wo_ref, bias_ref, out_ref,
             send_buf, red_ref, rs_buf, ag_buf,
             rs_send, rs_recv, ag_send, ag_recv):
        my = lax.axis_index("i")

        barrier_sem = pltpu.get_barrier_semaphore()
        for r in (1, 2, 3):
            pl.semaphore_signal(
                barrier_sem, inc=1,
                device_id=((my + r) % N_DEV,),
                device_id_type=pl.DeviceIdType.MESH,
            )
        pl.semaphore_wait(barrier_sem, 3)

        def compute_quarter(t):
            x_q = x_ref[pl.ds(t * M, M), :]
            q_q = jnp.dot(
                x_q, wq_ref[...], preferred_element_type=jnp.float32
            ).astype(bf16)
            b = t // 2
            kb_full = k_ref[b]
            vb_full = v_ref[b]
            bias_q = bias_ref[t % 2]
            ctx_parts = []
            for h in range(H_loc):
                cols = slice(h * Dh, (h + 1) * Dh)
                s = lax.dot_general(
                    q_q[:, cols], kb_full[:, cols],
                    (((1,), (1,)), ((), ())),
                    preferred_element_type=jnp.float32,
                )
                e = jnp.exp(s + bias_q)
                r = 1.0 / jnp.sum(e, axis=1, keepdims=True)
                ctx_h = jnp.dot(
                    e.astype(bf16), vb_full[:, cols],
                    preferred_element_type=jnp.float32,
                )
                ctx_parts.append(ctx_h * r)
            ctx = jnp.concatenate(ctx_parts, axis=1).astype(bf16)
            return jnp.dot(
                ctx, wo_ref[...], preferred_element_type=jnp.float32
            )

        rs_desc = []
        for r in (1, 2, 3):
            tgt = (my + r) % N_DEV
            send_buf[r - 1, :, :] = compute_quarter(tgt).astype(bf16)
            d = pltpu.make_async_remote_copy(
                src_ref=send_buf.at[r - 1],
                dst_ref=rs_buf.at[3 - r],
                send_sem=rs_send.at[r - 1],
                recv_sem=rs_recv.at[3 - r],
                device_id=(tgt,),
                device_id_type=pl.DeviceIdType.MESH,
            )
            d.start()
            rs_desc.append(d)

        acc = compute_quarter(my)
        for s_ in range(3):
            recv = pltpu.make_async_remote_copy(
                src_ref=rs_buf.at[s_], dst_ref=rs_buf.at[s_],
                send_sem=rs_send.at[0], recv_sem=rs_recv.at[s_],
                device_id=(my,), device_id_type=pl.DeviceIdType.MESH,
            )
            recv.wait_recv()
            acc = acc + rs_buf[s_].astype(jnp.float32)

        red_ref[...] = acc.astype(bf16)
        out_ref[pl.ds(my * M, M), :] = red_ref[...]

        ag_desc = []
        for r in (1, 2, 3):
            tgt = (my + r) % N_DEV
            d = pltpu.make_async_remote_copy(
                src_ref=red_ref,
                dst_ref=ag_buf.at[3 - r],
                send_sem=ag_send.at[r - 1],
                recv_sem=ag_recv.at[3 - r],
                device_id=(tgt,),
                device_id_type=pl.DeviceIdType.MESH,
            )
            d.start()
            ag_desc.append(d)

        for d in rs_desc:
            d.wait_send()

        for q in (1, 2, 3):
            sender = (my + q) % N_DEV
            recv = pltpu.make_async_remote_copy(
                src_ref=ag_buf.at[q - 1], dst_ref=ag_buf.at[q - 1],
                send_sem=ag_send.at[0], recv_sem=ag_recv.at[q - 1],
                device_id=(my,), device_id_type=pl.DeviceIdType.MESH,
            )
            recv.wait_recv()
            out_ref[pl.ds(sender * M, M), :] = ag_buf[q - 1]

        for d in ag_desc:
            d.wait_send()

    out = pl.pallas_call(
        body,
        out_shape=jax.ShapeDtypeStruct((B * Sq, d_model), bf16),
        in_specs=[pl.BlockSpec(memory_space=pltpu.VMEM)] * 6,
        out_specs=pl.BlockSpec(memory_space=pltpu.VMEM),
        scratch_shapes=[
            pltpu.VMEM((3, M, d_model), bf16),
            pltpu.VMEM((M, d_model), bf16),
            pltpu.VMEM((3, M, d_model), bf16),
            pltpu.VMEM((3, M, d_model), bf16),
            pltpu.SemaphoreType.DMA((3,)),
            pltpu.SemaphoreType.DMA((3,)),
            pltpu.SemaphoreType.DMA((3,)),
            pltpu.SemaphoreType.DMA((3,)),
        ],
        compiler_params=pltpu.CompilerParams(collective_id=0),
    )(xb, (Wq_loc * 0.125).astype(bf16), K2, V2, Wo_loc.astype(bf16), bias)

    return out.reshape(B, Sq, d_model)


# device time: 35492 ns/iter; 2.2228x vs baseline; 1.0487x over previous
import jax
import jax.numpy as jnp
import numpy as np
from jax import lax
from jax.experimental import pallas as pl
from jax.experimental.pallas import tpu as pltpu

N_DEV = 4


def _mask_bias(Sq, Skv):
    qb = (np.arange(Sq) // 64)[:, None]
    kb = (np.arange(Skv) // 64)[None, :]
    mask = (qb == kb) | (kb == 0) | ((qb + kb) % 3 == 0)
    bias = np.where(mask, 0.0, -30000.0).astype(np.float32)
    return bias.reshape(2, Sq // 2, Skv)


def kernel(x, Wq, K_ext, V_ext, Wo):
    B, Sq, d_model = x.shape
    _, Skv, H_loc, Dh = K_ext.shape
    d_loc = H_loc * Dh
    M = (B * Sq) // N_DEV
    Mh = M // 2

    bf16 = jnp.bfloat16
    my_out = lax.axis_index("i")

    Wq_loc = lax.dynamic_slice_in_dim(Wq, my_out * d_loc, d_loc, axis=1)
    Wo_loc = lax.dynamic_slice_in_dim(Wo, my_out * d_loc, d_loc, axis=0)

    xb = x.reshape(B * Sq, d_model).astype(bf16)
    K2 = K_ext.reshape(B, Skv, d_loc).astype(bf16)
    V2 = V_ext.reshape(B, Skv, d_loc).astype(bf16)
    bias = jnp.asarray(_mask_bias(Sq, Skv))

    def body(x_ref, wq_ref, k_ref, v_ref, wo_ref, bias_ref, out_ref,
             send_buf, red_ref, rs_buf, ag_buf,
             rs_send, rs_recv, ag_send, ag_recv):
        my = lax.axis_index("i")

        barrier_sem = pltpu.get_barrier_semaphore()
        for r in (1, 2, 3):
            pl.semaphore_signal(
                barrier_sem, inc=1,
                device_id=((my + r) % N_DEV,),
                device_id_type=pl.DeviceIdType.MESH,
            )
        pl.semaphore_wait(barrier_sem, 3)

        def compute_quarter(t):
            x_q = x_ref[pl.ds(t * M, M), :]
            q_q = jnp.dot(
                x_q, wq_ref[...], preferred_element_type=jnp.float32
            ).astype(bf16)
            b = t // 2
            kb_full = k_ref[b]
            vb_full = v_ref[b]
            bias_q = bias_ref[t % 2]
            ctx_parts = []
            for h in range(H_loc):
                cols = slice(h * Dh, (h + 1) * Dh)
                s = lax.dot_general(
                    q_q[:, cols], kb_full[:, cols],
                    (((1,), (1,)), ((), ())),
                    preferred_element_type=jnp.float32,
                )
                e = jnp.exp(s + bias_q)
                r = 1.0 / jnp.sum(e, axis=1, keepdims=True)
                ctx_h = jnp.dot(
                    e.astype(bf16), vb_full[:, cols],
                    preferred_element_type=jnp.float32,
                )
                ctx_parts.append(ctx_h * r)
            ctx = jnp.concatenate(ctx_parts, axis=1).astype(bf16)
            return jnp.dot(
                ctx, wo_ref[...], preferred_element_type=jnp.float32
            )

        rs_desc = []
        for r in (1, 2, 3):
            tgt = (my + r) % N_DEV
            part = compute_quarter(tgt).astype(bf16)
            send_buf[r - 1, 0, :, :] = part[:Mh, :]
            send_buf[r - 1, 1, :, :] = part[Mh:, :]
            for hf in (0, 1):
                d = pltpu.make_async_remote_copy(
                    src_ref=send_buf.at[r - 1, hf],
                    dst_ref=rs_buf.at[3 - r, hf],
                    send_sem=rs_send.at[(r - 1) * 2 + hf],
                    recv_sem=rs_recv.at[(3 - r) * 2 + hf],
                    device_id=(tgt,),
                    device_id_type=pl.DeviceIdType.MESH,
                )
                d.start()
                rs_desc.append(d)

        own = compute_quarter(my)

        ag_desc = []
        for hf in (0, 1):
            acc = own[hf * Mh:(hf + 1) * Mh, :]
            for s_ in range(3):
                recv = pltpu.make_async_remote_copy(
                    src_ref=rs_buf.at[s_, hf], dst_ref=rs_buf.at[s_, hf],
                    send_sem=rs_send.at[0], recv_sem=rs_recv.at[s_ * 2 + hf],
                    device_id=(my,), device_id_type=pl.DeviceIdType.MESH,
                )
                recv.wait_recv()
                acc = acc + rs_buf[s_, hf].astype(jnp.float32)

            red_ref[hf, :, :] = acc.astype(bf16)
            out_ref[pl.ds(my * M + hf * Mh, Mh), :] = red_ref[hf]

            for r in (1, 2, 3):
                tgt = (my + r) % N_DEV
                d = pltpu.make_async_remote_copy(
                    src_ref=red_ref.at[hf],
                    dst_ref=ag_buf.at[3 - r, hf],
                    send_sem=ag_send.at[(r - 1) * 2 + hf],
                    recv_sem=ag_recv.at[(3 - r) * 2 + hf],
                    device_id=(tgt,),
                    device_id_type=pl.DeviceIdType.MESH,
                )
                d.start()
                ag_desc.append(d)

        for d in rs_desc:
            d.wait_send()

        for hf in (0, 1):
            for q in (1, 2, 3):
                sender = (my + q) % N_DEV
                recv = pltpu.make_async_remote_copy(
                    src_ref=ag_buf.at[q - 1, hf], dst_ref=ag_buf.at[q - 1, hf],
                    send_sem=ag_send.at[0], recv_sem=ag_recv.at[(q - 1) * 2 + hf],
                    device_id=(my,), device_id_type=pl.DeviceIdType.MESH,
                )
                recv.wait_recv()
                out_ref[pl.ds(sender * M + hf * Mh, Mh), :] = ag_buf[q - 1, hf]

        for d in ag_desc:
            d.wait_send()

    out = pl.pallas_call(
        body,
        out_shape=jax.ShapeDtypeStruct((B * Sq, d_model), bf16),
        in_specs=[pl.BlockSpec(memory_space=pltpu.VMEM)] * 6,
        out_specs=pl.BlockSpec(memory_space=pltpu.VMEM),
        scratch_shapes=[
            pltpu.VMEM((3, 2, Mh, d_model), bf16),
            pltpu.VMEM((2, Mh, d_model), bf16),
            pltpu.VMEM((3, 2, Mh, d_model), bf16),
            pltpu.VMEM((3, 2, Mh, d_model), bf16),
            pltpu.SemaphoreType.DMA((6,)),
            pltpu.SemaphoreType.DMA((6,)),
            pltpu.SemaphoreType.DMA((6,)),
            pltpu.SemaphoreType.DMA((6,)),
        ],
        compiler_params=pltpu.CompilerParams(collective_id=0),
    )(xb, (Wq_loc * 0.125).astype(bf16), K2, V2, Wo_loc.astype(bf16), bias)

    return out.reshape(B, Sq, d_model)


# device time: 35478 ns/iter; 2.2237x vs baseline; 1.0004x over previous
import jax
import jax.numpy as jnp
import numpy as np
from jax import lax
from jax.experimental import pallas as pl
from jax.experimental.pallas import tpu as pltpu

N_DEV = 4


def _mask_bias(Sq, Skv):
    qb = (np.arange(Sq) // 64)[:, None]
    kb = (np.arange(Skv) // 64)[None, :]
    mask = (qb == kb) | (kb == 0) | ((qb + kb) % 3 == 0)
    bias = np.where(mask, 0.0, -30000.0).astype(np.float32)
    return bias.reshape(2, Sq // 2, Skv)


def kernel(x, Wq, K_ext, V_ext, Wo):
    B, Sq, d_model = x.shape
    _, Skv, H_loc, Dh = K_ext.shape
    d_loc = H_loc * Dh
    M = (B * Sq) // N_DEV
    Mh = M // 2

    bf16 = jnp.bfloat16
    my_out = lax.axis_index("i")

    Wq_loc = lax.dynamic_slice_in_dim(Wq, my_out * d_loc, d_loc, axis=1)
    Wo_loc = lax.dynamic_slice_in_dim(Wo, my_out * d_loc, d_loc, axis=0)

    xb = x.reshape(B * Sq, d_model).astype(bf16)
    K2 = K_ext.reshape(B, Skv, d_loc).astype(bf16)
    V2 = V_ext.reshape(B, Skv, d_loc).astype(bf16)
    bias = jnp.asarray(_mask_bias(Sq, Skv))

    def body(x_ref, wq_ref, k_ref, v_ref, wo_ref, bias_ref, out_ref,
             send_buf, red_ref, rs_buf,
             rs_send, rs_recv, ag_send, ag_recv):
        my = lax.axis_index("i")

        barrier_sem = pltpu.get_barrier_semaphore()
        for r in (1, 2, 3):
            pl.semaphore_signal(
                barrier_sem, inc=1,
                device_id=((my + r) % N_DEV,),
                device_id_type=pl.DeviceIdType.MESH,
            )
        pl.semaphore_wait(barrier_sem, 3)

        def compute_quarter(t):
            x_q = x_ref[pl.ds(t * M, M), :]
            q_q = jnp.dot(
                x_q, wq_ref[...], preferred_element_type=jnp.float32
            ).astype(bf16)
            b = t // 2
            kb_full = k_ref[b]
            vb_full = v_ref[b]
            bias_q = bias_ref[t % 2]
            ctx_parts = []
            for h in range(H_loc):
                cols = slice(h * Dh, (h + 1) * Dh)
                s = lax.dot_general(
                    q_q[:, cols], kb_full[:, cols],
                    (((1,), (1,)), ((), ())),
                    preferred_element_type=jnp.float32,
                )
                e = jnp.exp(s + bias_q)
                r = 1.0 / jnp.sum(e, axis=1, keepdims=True)
                ctx_h = jnp.dot(
                    e.astype(bf16), vb_full[:, cols],
                    preferred_element_type=jnp.float32,
                )
                ctx_parts.append(ctx_h * r)
            ctx = jnp.concatenate(ctx_parts, axis=1).astype(bf16)
            return jnp.dot(
                ctx, wo_ref[...], preferred_element_type=jnp.float32
            )

        rs_desc = []
        for r in (1, 2, 3):
            tgt = (my + r) % N_DEV
            part = compute_quarter(tgt).astype(bf16)
            send_buf[r - 1, 0, :, :] = part[:Mh, :]
            send_buf[r - 1, 1, :, :] = part[Mh:, :]
            for hf in (0, 1):
                d = pltpu.make_async_remote_copy(
                    src_ref=send_buf.at[r - 1, hf],
                    dst_ref=rs_buf.at[3 - r, hf],
                    send_sem=rs_send.at[(r - 1) * 2 + hf],
                    recv_sem=rs_recv.at[(3 - r) * 2 + hf],
                    device_id=(tgt,),
                    device_id_type=pl.DeviceIdType.MESH,
                )
                d.start()
                rs_desc.append(d)

        own = compute_quarter(my)

        ag_desc = []
        for hf in (0, 1):
            acc = own[hf * Mh:(hf + 1) * Mh, :]
            for s_ in range(3):
                recv = pltpu.make_async_remote_copy(
                    src_ref=rs_buf.at[s_, hf], dst_ref=rs_buf.at[s_, hf],
                    send_sem=rs_send.at[0], recv_sem=rs_recv.at[s_ * 2 + hf],
                    device_id=(my,), device_id_type=pl.DeviceIdType.MESH,
                )
                recv.wait_recv()
                acc = acc + rs_buf[s_, hf].astype(jnp.float32)

            red_ref[hf, :, :] = acc.astype(bf16)
            out_ref[pl.ds(my * M + hf * Mh, Mh), :] = red_ref[hf]

            for r in (1, 2, 3):
                tgt = (my + r) % N_DEV
                d = pltpu.make_async_remote_copy(
                    src_ref=red_ref.at[hf],
                    dst_ref=out_ref.at[pl.ds(my * M + hf * Mh, Mh)],
                    send_sem=ag_send.at[(r - 1) * 2 + hf],
                    recv_sem=ag_recv.at[(3 - r) * 2 + hf],
                    device_id=(tgt,),
                    device_id_type=pl.DeviceIdType.MESH,
                )
                d.start()
                ag_desc.append(d)

        for d in rs_desc:
            d.wait_send()

        for q in (1, 2, 3):
            sender = (my + q) % N_DEV
            for hf in (0, 1):
                recv = pltpu.make_async_remote_copy(
                    src_ref=red_ref.at[hf],
                    dst_ref=out_ref.at[pl.ds(sender * M + hf * Mh, Mh)],
                    send_sem=ag_send.at[0],
                    recv_sem=ag_recv.at[(q - 1) * 2 + hf],
                    device_id=(my,), device_id_type=pl.DeviceIdType.MESH,
                )
                recv.wait_recv()

        for d in ag_desc:
            d.wait_send()

    out = pl.pallas_call(
        body,
        out_shape=jax.ShapeDtypeStruct((B * Sq, d_model), bf16),
        in_specs=[pl.BlockSpec(memory_space=pltpu.VMEM)] * 6,
        out_specs=pl.BlockSpec(memory_space=pltpu.VMEM),
        scratch_shapes=[
            pltpu.VMEM((3, 2, Mh, d_model), bf16),
            pltpu.VMEM((2, Mh, d_model), bf16),
            pltpu.VMEM((3, 2, Mh, d_model), bf16),
            pltpu.SemaphoreType.DMA((6,)),
            pltpu.SemaphoreType.DMA((6,)),
            pltpu.SemaphoreType.DMA((6,)),
            pltpu.SemaphoreType.DMA((6,)),
        ],
        compiler_params=pltpu.CompilerParams(collective_id=0),
    )(xb, (Wq_loc * 0.125).astype(bf16), K2, V2, Wo_loc.astype(bf16), bias)

    return out.reshape(B, Sq, d_model)
